# Initial kernel scaffold; baseline (speedup 1.0000x reference)
import jax
import jax.numpy as jnp
from jax.experimental import pallas as pl


def kernel(node_features, edge_features, adj, weight, bias):
    raise NotImplementedError("write your pallas kernel here")


# speedup vs baseline: 1.4798x; 1.4798x over previous
import functools

import jax
import jax.numpy as jnp
from jax.experimental import pallas as pl
from jax.experimental.pallas import tpu as pltpu


def _fused_kernel(nf_i_ref, adj_ref, e_ref, nf_j_ref, w1_ref, w2_ref,
                  w3_ref, b_ref, out_ref, acc_nn, acc_ne):
    j = pl.program_id(1)
    nj = pl.num_programs(1)

    @pl.when(j == 0)
    def _init():
        acc_nn[...] = jnp.zeros_like(acc_nn)
        acc_ne[...] = jnp.zeros_like(acc_ne)

    a = adj_ref[...]
    acc_nn[...] += jnp.dot(a, nf_j_ref[...], preferred_element_type=jnp.float32)
    acc_ne[...] += jnp.sum(a * e_ref[...].T, axis=1, keepdims=True)

    @pl.when(j == nj - 1)
    def _combine():
        out_ref[...] = (
            jnp.dot(nf_i_ref[...], w1_ref[...], preferred_element_type=jnp.float32)
            + jnp.dot(acc_nn[...], w2_ref[...], preferred_element_type=jnp.float32)
            + acc_ne[...] * w3_ref[...]
            + b_ref[...]
        )


@functools.partial(jax.jit, static_argnames=("bm", "bk", "interpret"))
def _readout(node_features, edge_features, adj, weight, bias,
             bm=512, bk=512, interpret=False):
    n, d = node_features.shape
    out_dim = weight.shape[1]
    w1 = weight[:d]
    w2 = weight[d:2 * d]
    w3 = weight[2 * d:2 * d + 1]
    b = bias.reshape(1, out_dim)
    grid = (n // bm, n // bk)
    return pl.pallas_call(
        _fused_kernel,
        grid=grid,
        in_specs=[
            pl.BlockSpec((bm, d), lambda i, j: (i, 0)),
            pl.BlockSpec((bm, bk), lambda i, j: (i, j)),
            pl.BlockSpec((bk, bm), lambda i, j: (j, i)),
            pl.BlockSpec((bk, d), lambda i, j: (j, 0)),
            pl.BlockSpec((d, out_dim), lambda i, j: (0, 0)),
            pl.BlockSpec((d, out_dim), lambda i, j: (0, 0)),
            pl.BlockSpec((1, out_dim), lambda i, j: (0, 0)),
            pl.BlockSpec((1, out_dim), lambda i, j: (0, 0)),
        ],
        out_specs=pl.BlockSpec((bm, out_dim), lambda i, j: (i, 0)),
        out_shape=jax.ShapeDtypeStruct((n, out_dim), jnp.float32),
        scratch_shapes=[
            pltpu.VMEM((bm, out_dim), jnp.float32),
            pltpu.VMEM((bm, 1), jnp.float32),
        ],
        compiler_params=pltpu.CompilerParams(
            dimension_semantics=("parallel", "arbitrary"),
        ),
        interpret=interpret,
    )(node_features, adj, edge_features, node_features, w1, w2, w3, b)


def kernel(node_features, edge_features, adj, weight, bias):
    return _readout(node_features, edge_features, adj, weight, bias)


# speedup vs baseline: 1.5161x; 1.0246x over previous
import functools

import jax
import jax.numpy as jnp
from jax.experimental import pallas as pl
from jax.experimental.pallas import tpu as pltpu


def _fused_kernel(nf_ref, adj_ref, e_ref, w1_ref, w2_ref,
                  w3_ref, b_ref, out_ref, acc_nn, acc_ne, *, bm, bk):
    i = pl.program_id(0)
    j = pl.program_id(1)
    nj = pl.num_programs(1)

    @pl.when(j == 0)
    def _init():
        acc_nn[...] = jnp.zeros_like(acc_nn)
        acc_ne[...] = jnp.zeros_like(acc_ne)

    a = adj_ref[...]
    nf_j = nf_ref[pl.ds(j * bk, bk), :]
    acc_nn[...] += jnp.dot(a, nf_j, preferred_element_type=jnp.float32)
    acc_ne[...] += jnp.sum(a * e_ref[...].T, axis=1, keepdims=True)

    @pl.when(j == nj - 1)
    def _combine():
        nf_i = nf_ref[pl.ds(i * bm, bm), :]
        out_ref[...] = (
            jnp.dot(nf_i, w1_ref[...], preferred_element_type=jnp.float32)
            + jnp.dot(acc_nn[...], w2_ref[...], preferred_element_type=jnp.float32)
            + acc_ne[...] * w3_ref[...]
            + b_ref[...]
        )


@functools.partial(jax.jit, static_argnames=("bm", "bk", "interpret"))
def _readout(node_features, edge_features, adj, weight, bias,
             bm=512, bk=512, interpret=False):
    n, d = node_features.shape
    out_dim = weight.shape[1]
    w1 = weight[:d]
    w2 = weight[d:2 * d]
    w3 = weight[2 * d:2 * d + 1]
    b = bias.reshape(1, out_dim)
    grid = (n // bm, n // bk)
    return pl.pallas_call(
        functools.partial(_fused_kernel, bm=bm, bk=bk),
        grid=grid,
        in_specs=[
            pl.BlockSpec((n, d), lambda i, j: (0, 0)),
            pl.BlockSpec((bm, bk), lambda i, j: (i, j)),
            pl.BlockSpec((bk, bm), lambda i, j: (j, i)),
            pl.BlockSpec((d, out_dim), lambda i, j: (0, 0)),
            pl.BlockSpec((d, out_dim), lambda i, j: (0, 0)),
            pl.BlockSpec((1, out_dim), lambda i, j: (0, 0)),
            pl.BlockSpec((1, out_dim), lambda i, j: (0, 0)),
        ],
        out_specs=pl.BlockSpec((bm, out_dim), lambda i, j: (i, 0)),
        out_shape=jax.ShapeDtypeStruct((n, out_dim), jnp.float32),
        scratch_shapes=[
            pltpu.VMEM((bm, out_dim), jnp.float32),
            pltpu.VMEM((bm, 1), jnp.float32),
        ],
        compiler_params=pltpu.CompilerParams(
            dimension_semantics=("parallel", "arbitrary"),
        ),
        interpret=interpret,
    )(node_features, adj, edge_features, w1, w2, w3, b)


def kernel(node_features, edge_features, adj, weight, bias):
    return _readout(node_features, edge_features, adj, weight, bias)


# speedup vs baseline: 2.1204x; 1.3986x over previous
import functools

import jax
import jax.numpy as jnp
from jax.experimental import pallas as pl
from jax.experimental.pallas import tpu as pltpu


def _fused_kernel(nf_ref, adj_ref, e_ref, w1_ref, w2_ref,
                  w3_ref, b_ref, out_ref, acc_nn, acc_ne, *, bm, bk):
    i = pl.program_id(0)
    j = pl.program_id(1)
    nj = pl.num_programs(1)

    @pl.when(j == 0)
    def _init():
        acc_nn[...] = jnp.zeros_like(acc_nn)
        acc_ne[...] = jnp.zeros_like(acc_ne)

    a = adj_ref[...]
    nf_j = nf_ref[pl.ds(j * bk, bk), :]
    acc_nn[...] += jnp.dot(a, nf_j, preferred_element_type=jnp.float32)
    acc_ne[...] += jnp.sum(a * e_ref[...].T, axis=1, keepdims=True)

    @pl.when(j == nj - 1)
    def _combine():
        nf_i = nf_ref[pl.ds(i * bm, bm), :]
        out_ref[...] = (
            jnp.dot(nf_i, w1_ref[...], preferred_element_type=jnp.float32)
            + jnp.dot(acc_nn[...], w2_ref[...], preferred_element_type=jnp.float32)
            + acc_ne[...] * w3_ref[...]
            + b_ref[...]
        )


@functools.partial(jax.jit, static_argnames=("bm", "bk", "interpret"))
def _readout(node_features, edge_features, adj, weight, bias,
             bm=512, bk=1024, interpret=False):
    n, d = node_features.shape
    out_dim = weight.shape[1]
    w1 = weight[:d]
    w2 = weight[d:2 * d]
    w3 = weight[2 * d:2 * d + 1]
    b = bias.reshape(1, out_dim)
    grid = (n // bm, n // bk)
    return pl.pallas_call(
        functools.partial(_fused_kernel, bm=bm, bk=bk),
        grid=grid,
        in_specs=[
            pl.BlockSpec((n, d), lambda i, j: (0, 0)),
            pl.BlockSpec((bm, bk), lambda i, j: (i, j)),
            pl.BlockSpec((bk, bm), lambda i, j: (j, i)),
            pl.BlockSpec((d, out_dim), lambda i, j: (0, 0)),
            pl.BlockSpec((d, out_dim), lambda i, j: (0, 0)),
            pl.BlockSpec((1, out_dim), lambda i, j: (0, 0)),
            pl.BlockSpec((1, out_dim), lambda i, j: (0, 0)),
        ],
        out_specs=pl.BlockSpec((bm, out_dim), lambda i, j: (i, 0)),
        out_shape=jax.ShapeDtypeStruct((n, out_dim), jnp.float32),
        scratch_shapes=[
            pltpu.VMEM((bm, out_dim), jnp.float32),
            pltpu.VMEM((bm, 1), jnp.float32),
        ],
        compiler_params=pltpu.CompilerParams(
            dimension_semantics=("parallel", "arbitrary"),
        ),
        interpret=interpret,
    )(node_features, adj, edge_features, w1, w2, w3, b)


def kernel(node_features, edge_features, adj, weight, bias):
    return _readout(node_features, edge_features, adj, weight, bias)


# speedup vs baseline: 2.5923x; 1.2226x over previous
import functools

import jax
import jax.numpy as jnp
from jax.experimental import pallas as pl
from jax.experimental.pallas import tpu as pltpu


def _fused_kernel(nf_ref, adj_ref, e_ref, w1_ref, w2_ref,
                  w3_ref, b_ref, out_ref, acc_nn, acc_ne, *, bm, bk):
    i = pl.program_id(0)
    j = pl.program_id(1)
    nj = pl.num_programs(1)

    @pl.when(j == 0)
    def _init():
        acc_nn[...] = jnp.zeros_like(acc_nn)
        acc_ne[...] = jnp.zeros_like(acc_ne)

    a = adj_ref[...]
    nf_j = nf_ref[pl.ds(j * bk, bk), :]
    acc_nn[...] += jnp.dot(a, nf_j, preferred_element_type=jnp.float32)
    acc_ne[...] += jnp.sum(a * e_ref[...].T, axis=1, keepdims=True)

    @pl.when(j == nj - 1)
    def _combine():
        nf_i = nf_ref[pl.ds(i * bm, bm), :]
        out_ref[...] = (
            jnp.dot(nf_i, w1_ref[...], preferred_element_type=jnp.float32)
            + jnp.dot(acc_nn[...], w2_ref[...], preferred_element_type=jnp.float32)
            + acc_ne[...] * w3_ref[...]
            + b_ref[...]
        )


@functools.partial(jax.jit, static_argnames=("bm", "bk", "interpret"))
def _readout(node_features, edge_features, adj, weight, bias,
             bm=512, bk=2048, interpret=False):
    n, d = node_features.shape
    out_dim = weight.shape[1]
    w1 = weight[:d]
    w2 = weight[d:2 * d]
    w3 = weight[2 * d:2 * d + 1]
    b = bias.reshape(1, out_dim)
    grid = (n // bm, n // bk)
    return pl.pallas_call(
        functools.partial(_fused_kernel, bm=bm, bk=bk),
        grid=grid,
        in_specs=[
            pl.BlockSpec((n, d), lambda i, j: (0, 0)),
            pl.BlockSpec((bm, bk), lambda i, j: (i, j)),
            pl.BlockSpec((bk, bm), lambda i, j: (j, i)),
            pl.BlockSpec((d, out_dim), lambda i, j: (0, 0)),
            pl.BlockSpec((d, out_dim), lambda i, j: (0, 0)),
            pl.BlockSpec((1, out_dim), lambda i, j: (0, 0)),
            pl.BlockSpec((1, out_dim), lambda i, j: (0, 0)),
        ],
        out_specs=pl.BlockSpec((bm, out_dim), lambda i, j: (i, 0)),
        out_shape=jax.ShapeDtypeStruct((n, out_dim), jnp.float32),
        scratch_shapes=[
            pltpu.VMEM((bm, out_dim), jnp.float32),
            pltpu.VMEM((bm, 1), jnp.float32),
        ],
        compiler_params=pltpu.CompilerParams(
            dimension_semantics=("parallel", "arbitrary"),
        ),
        interpret=interpret,
    )(node_features, adj, edge_features, w1, w2, w3, b)


def kernel(node_features, edge_features, adj, weight, bias):
    return _readout(node_features, edge_features, adj, weight, bias)


# speedup vs baseline: 2.6101x; 1.0068x over previous
import functools

import jax
import jax.numpy as jnp
from jax.experimental import pallas as pl
from jax.experimental.pallas import tpu as pltpu


def _fused_kernel(nf_ref, adj_ref, e_ref, w1_ref, w2_ref,
                  w3_ref, b_ref, out_ref, acc_nn, acc_ne, *, bm, bk):
    i = pl.program_id(0)
    j = pl.program_id(1)
    nj = pl.num_programs(1)

    @pl.when(j == 0)
    def _init():
        acc_nn[...] = jnp.zeros_like(acc_nn)
        acc_ne[...] = jnp.zeros_like(acc_ne)

    a = adj_ref[...]
    nf_j = nf_ref[pl.ds(j * bk, bk), :]
    acc_nn[...] += jnp.dot(a, nf_j, preferred_element_type=jnp.float32)
    acc_ne[...] += jnp.sum(a * e_ref[...].T, axis=1, keepdims=True)

    @pl.when(j == nj - 1)
    def _combine():
        nf_i = nf_ref[pl.ds(i * bm, bm), :]
        out_ref[...] = (
            jnp.dot(nf_i, w1_ref[...], preferred_element_type=jnp.float32)
            + jnp.dot(acc_nn[...], w2_ref[...], preferred_element_type=jnp.float32)
            + acc_ne[...] * w3_ref[...]
            + b_ref[...]
        )


@functools.partial(jax.jit, static_argnames=("bm", "bk", "interpret"))
def _readout(node_features, edge_features, adj, weight, bias,
             bm=512, bk=4096, interpret=False):
    n, d = node_features.shape
    out_dim = weight.shape[1]
    w1 = weight[:d]
    w2 = weight[d:2 * d]
    w3 = weight[2 * d:2 * d + 1]
    b = bias.reshape(1, out_dim)
    grid = (n // bm, n // bk)
    return pl.pallas_call(
        functools.partial(_fused_kernel, bm=bm, bk=bk),
        grid=grid,
        in_specs=[
            pl.BlockSpec((n, d), lambda i, j: (0, 0)),
            pl.BlockSpec((bm, bk), lambda i, j: (i, j)),
            pl.BlockSpec((bk, bm), lambda i, j: (j, i)),
            pl.BlockSpec((d, out_dim), lambda i, j: (0, 0)),
            pl.BlockSpec((d, out_dim), lambda i, j: (0, 0)),
            pl.BlockSpec((1, out_dim), lambda i, j: (0, 0)),
            pl.BlockSpec((1, out_dim), lambda i, j: (0, 0)),
        ],
        out_specs=pl.BlockSpec((bm, out_dim), lambda i, j: (i, 0)),
        out_shape=jax.ShapeDtypeStruct((n, out_dim), jnp.float32),
        scratch_shapes=[
            pltpu.VMEM((bm, out_dim), jnp.float32),
            pltpu.VMEM((bm, 1), jnp.float32),
        ],
        compiler_params=pltpu.CompilerParams(
            dimension_semantics=("parallel", "arbitrary"),
        ),
        interpret=interpret,
    )(node_features, adj, edge_features, w1, w2, w3, b)


def kernel(node_features, edge_features, adj, weight, bias):
    return _readout(node_features, edge_features, adj, weight, bias)
